# Initial kernel scaffold; baseline (speedup 1.0000x reference)
import jax
import jax.numpy as jnp
from jax.experimental import pallas as pl


def kernel(x, graph_idx, device):
    raise NotImplementedError("write your pallas kernel here")


# speedup vs baseline: 13.6068x; 13.6068x over previous
import jax
import jax.numpy as jnp
from jax.experimental import pallas as pl

_BSZ = 32768
_NCLS = 512
_NGRAPH = 16


def _body(x_ref, idx_ref, var_ref, flag_ref):
    xs = x_ref[...]
    m = jnp.max(xs, axis=1, keepdims=True)
    s = jnp.sum(jnp.exp(xs - m), axis=1, keepdims=True)
    loss = m + jnp.log(s) - xs[:, 0:1]

    idx = idx_ref[...]
    ids = jax.lax.broadcasted_iota(jnp.int32, (_NGRAPH, _BSZ), 0)
    cnt = jnp.sum((idx == ids).astype(jnp.float32), axis=1, keepdims=True)
    p = (cnt > 0.0).astype(jnp.float32)

    n = jnp.sum(p)
    mean = jnp.sum(loss * p) / n
    var = jnp.sum(jnp.square(loss - mean) * p) / (n - 1.0)
    var_ref[...] = jnp.broadcast_to(var, (1, 1))
    flag_ref[...] = jnp.broadcast_to((n == 1.0).astype(jnp.int32), (1, 1))


def kernel(x, graph_idx, device):
    idx = graph_idx.astype(jnp.int32).reshape(1, _BSZ)
    var, flag = pl.pallas_call(
        _body,
        grid=(1,),
        in_specs=[
            pl.BlockSpec((_NGRAPH, _NCLS), lambda i: (0, 0)),
            pl.BlockSpec((1, _BSZ), lambda i: (0, 0)),
        ],
        out_specs=[
            pl.BlockSpec((1, 1), lambda i: (0, 0)),
            pl.BlockSpec((1, 1), lambda i: (0, 0)),
        ],
        out_shape=[
            jax.ShapeDtypeStruct((1, 1), jnp.float32),
            jax.ShapeDtypeStruct((1, 1), jnp.int32),
        ],
    )(x, idx)
    return var[0, 0], flag[0, 0] == 1
